# Initial kernel scaffold; baseline (speedup 1.0000x reference)
import functools

import jax
import jax.numpy as jnp
from jax import lax
from jax.experimental import pallas as pl
from jax.experimental.pallas import tpu as pltpu
from jax.experimental.pallas import tpu_sc as plsc

_B = 16384
_F = 26
_V = 100000
_D = 32
_N_NUM = 13
_OUT = _F * _D + _N_NUM

_info = plsc.get_sparse_core_info()
_NC, _NS = _info.num_cores, _info.num_subcores
_NW = _NC * _NS
_BPW = _B // _NW
_CHUNK = 128
_NCH = _BPW // _CHUNK


def _sc_gather_concat(flat_idx, num_features, tab_flat):
  mesh = plsc.VectorSubcoreMesh(core_axis_name="c", subcore_axis_name="s")

  @functools.partial(
      pl.kernel,
      mesh=mesh,
      out_type=jax.ShapeDtypeStruct((_B, _OUT), jnp.float32),
      scratch_types=[
          pltpu.VMEM((_NCH, _CHUNK), jnp.int32),
          pltpu.VMEM((_BPW, _D), jnp.float32),
          pltpu.SemaphoreType.DMA,
      ],
  )
  def k(idx_hbm, num_hbm, tab_hbm, out_hbm, idx_v, rows_v, sem):
    wid = lax.axis_index("s") * _NC + lax.axis_index("c")
    b0 = wid * _BPW
    w0 = wid * _NCH

    pltpu.sync_copy(
        num_hbm.at[pl.ds(b0, _BPW)],
        out_hbm.at[pl.ds(b0, _BPW), pl.ds(_F * _D, _N_NUM)])

    def body(f, carry):
      pltpu.sync_copy(idx_hbm.at[f, pl.ds(w0, _NCH)], idx_v)
      copies = [
          pltpu.async_copy(
              tab_hbm.at[idx_v.at[c]],
              rows_v.at[pl.ds(c * _CHUNK, _CHUNK)], sem)
          for c in range(_NCH)
      ]
      for cp in copies:
        cp.wait()
      pltpu.sync_copy(
          rows_v, out_hbm.at[pl.ds(b0, _BPW), pl.ds(f * _D, _D)])
      return carry

    lax.fori_loop(0, _F, body, 0)

  return k(flat_idx, num_features, tab_flat)


def kernel(cat_indices, num_features, tables):
  offs = (jnp.arange(_F, dtype=jnp.int32) * _V)[None, :]
  flat_idx = (cat_indices + offs).T.reshape(_F, _B // _CHUNK, _CHUNK)
  tab_flat = tables.reshape(_F * _V, _D)
  return _sc_gather_concat(flat_idx, num_features, tab_flat)


# speedup vs baseline: 1.1910x; 1.1910x over previous
import functools

import jax
import jax.numpy as jnp
from jax import lax
from jax.experimental import pallas as pl
from jax.experimental.pallas import tpu as pltpu
from jax.experimental.pallas import tpu_sc as plsc

_B = 16384
_F = 26
_V = 100000
_D = 32
_N_NUM = 13
_OUT = _F * _D + _N_NUM

_info = plsc.get_sparse_core_info()
_NC, _NS = _info.num_cores, _info.num_subcores
_NW = _NC * _NS
_BPW = _B // _NW
_CHUNK = 128
_NCH = _BPW // _CHUNK


def _sc_gather_concat(flat_idx, num_features, tab_flat):
  mesh = plsc.VectorSubcoreMesh(core_axis_name="c", subcore_axis_name="s")

  @functools.partial(
      pl.kernel,
      mesh=mesh,
      compiler_params=pltpu.CompilerParams(use_tc_tiling_on_sc=False),
      out_type=jax.ShapeDtypeStruct((_B, _OUT), jnp.float32),
      scratch_types=[
          pltpu.VMEM((_NCH, _CHUNK), jnp.int32),
          pltpu.VMEM((_NCH, _CHUNK), jnp.int32),
          pltpu.VMEM((_BPW, _D), jnp.float32),
          pltpu.VMEM((_BPW, _D), jnp.float32),
          pltpu.SemaphoreType.DMA,
          pltpu.SemaphoreType.DMA,
          pltpu.SemaphoreType.DMA,
          pltpu.SemaphoreType.DMA,
          pltpu.SemaphoreType.DMA,
      ],
  )
  def k(idx_hbm, num_hbm, tab_hbm, out_hbm,
        idx_a, idx_b, rows_a, rows_b, g_a, g_b, i_a, i_b, n_sem):
    wid = lax.axis_index("s") * _NC + lax.axis_index("c")
    b0 = wid * _BPW
    w0 = wid * _NCH

    num_cp = pltpu.async_copy(
        num_hbm.at[pl.ds(b0, _BPW)],
        out_hbm.at[pl.ds(b0, _BPW), pl.ds(_F * _D, _N_NUM)], n_sem)

    def fire(idx_v, rows_v, sem):
      return [
          pltpu.async_copy(
              tab_hbm.at[idx_v.at[c]],
              rows_v.at[pl.ds(c * _CHUNK, _CHUNK)], sem)
          for c in range(_NCH)
      ]

    def writeback(rows_v, f):
      pltpu.sync_copy(
          rows_v, out_hbm.at[pl.ds(b0, _BPW), pl.ds(f * _D, _D)])

    pltpu.sync_copy(idx_hbm.at[0, pl.ds(w0, _NCH)], idx_a)
    ga = fire(idx_a, rows_a, g_a)

    def body(i, carry):
      fa = 2 * i
      ib = pltpu.async_copy(idx_hbm.at[fa + 1, pl.ds(w0, _NCH)], idx_b, i_b)
      for cp in ga:
        cp.wait()
      ib.wait()
      gb = fire(idx_b, rows_b, g_b)
      writeback(rows_a, fa)

      @pl.when(i < _F // 2 - 1)
      def _():
        pltpu.async_copy(idx_hbm.at[fa + 2, pl.ds(w0, _NCH)], idx_a, i_a)

      for cp in gb:
        cp.wait()

      @pl.when(i < _F // 2 - 1)
      def _():
        pltpu.make_async_copy(idx_hbm.at[0, pl.ds(w0, _NCH)], idx_a, i_a).wait()
        fire(idx_a, rows_a, g_a)
      writeback(rows_b, fa + 1)
      return carry

    lax.fori_loop(0, _F // 2, body, 0)
    num_cp.wait()

  return k(flat_idx, num_features, tab_flat)


def kernel(cat_indices, num_features, tables):
  offs = (jnp.arange(_F, dtype=jnp.int32) * _V)[None, :]
  flat_idx = (cat_indices + offs).T.reshape(_F, _B // _CHUNK, _CHUNK)
  tab_flat = tables.reshape(_F * _V, _D)
  return _sc_gather_concat(flat_idx, num_features, tab_flat)
